# baseline (speedup 1.0000x reference)
import functools

import jax
import jax.numpy as jnp
from jax import lax
from jax.experimental import pallas as pl
from jax.experimental.pallas import tpu as pltpu
from jax.experimental.pallas import tpu_sc as plsc

N = 10000
E = 320000
D = 128
DE = 16
C = 128

NC = 2
NS = 16
NW = NC * NS
EB = 40
EB2 = 40
E_PER_TILE = E // NW
NBLK = E_PER_TILE // EB
NBLK2 = E_PER_TILE // EB2
ROWS_PER_SUB = 625
QTB = 160
A2W = 32
INV_SQRT_C = 1.0 / (C ** 0.5)

_R = 400
_GRID = N // _R


def _pre_body(x_r, wq_r, bq_r, wk_r, bk_r, wv_r, bv_r, wet_r, wsk_r, bsk_r,
              qt_r, kv_r, sk_r):
    xb = x_r[...]
    q = jnp.dot(xb, wq_r[...], preferred_element_type=jnp.float32) + bq_r[...]
    qt_r[:, :D] = q.astype(jnp.bfloat16)
    qt_r[:, D:D + DE] = jnp.dot(
        q, wet_r[...], preferred_element_type=jnp.float32).astype(jnp.bfloat16)
    qt_r[:, D + DE:] = jnp.zeros((_R, QTB - D - DE), jnp.bfloat16)
    kv_r[:, :D] = (jnp.dot(xb, wk_r[...], preferred_element_type=jnp.float32)
                   + bk_r[...]).astype(jnp.bfloat16)
    kv_r[:, D:] = (jnp.dot(xb, wv_r[...], preferred_element_type=jnp.float32)
                   + bv_r[...]).astype(jnp.bfloat16)
    sk_r[...] = jnp.dot(xb, wsk_r[...], preferred_element_type=jnp.float32) + bsk_r[...]


def _tc_pre(x, Wq, bq, Wk, bk, Wv, bv, WeT, Wskip, bskip):
    full = lambda shape: pl.BlockSpec(shape, lambda i: (0,) * len(shape))
    return pl.pallas_call(
        _pre_body,
        grid=(_GRID,),
        in_specs=[
            pl.BlockSpec((_R, D), lambda i: (i, 0)),
            full((D, C)), full((1, C)),
            full((D, C)), full((1, C)),
            full((D, C)), full((1, C)),
            full((D, DE)),
            full((D, C)), full((1, C)),
        ],
        out_specs=[
            pl.BlockSpec((_R, QTB), lambda i: (i, 0)),
            pl.BlockSpec((_R, 2 * D), lambda i: (i, 0)),
            pl.BlockSpec((_R, C), lambda i: (i, 0)),
        ],
        out_shape=[
            jax.ShapeDtypeStruct((N, QTB), jnp.bfloat16),
            jax.ShapeDtypeStruct((N, 2 * D), jnp.bfloat16),
            jax.ShapeDtypeStruct((N, C), jnp.float32),
        ],
    )(x, Wq, bq, Wk, bk, Wv, bv, WeT, Wskip, bskip)


def _sc1_body(qt_hbm, kv_hbm, ea_hbm, src_hbm, dst_hbm, acc_hbm, s_hbm,
              src_i, dst_i, qt0, qt1, kv0, kv1, ea0, ea1, st0, st1, sv0, sv1,
              acc_sh, sem_g0, sem_g1, sem_s0, sem_s1, sem_c0, sem_c1, sem_i):
    c = lax.axis_index("c")
    s = lax.axis_index("s")
    qt_b = (qt0, qt1)
    kv_b = (kv0, kv1)
    ea_b = (ea0, ea1)
    st_b = (st0, st1)
    sv_b = (sv0, sv1)
    sem_g = (sem_g0, sem_g1)
    sem_s = (sem_s0, sem_s1)
    sem_c = (sem_c0, sem_c1)

    zeros16 = jnp.zeros((16,), jnp.float32)
    lane = lax.iota(jnp.int32, 16)
    perms = [lane ^ m for m in (8, 4, 2, 1)]

    def hsum_splat(vec):
        for p in perms:
            vec = vec + jnp.take_along_axis(vec, p, axis=0,
                                            mode="promise_in_bounds")
        return vec

    def zero_row(r, _):
        for jj in range(D // 16):
            st0[r, pl.ds(16 * jj, 16)] = zeros16
        return 0

    lax.fori_loop(0, EB, zero_row, 0)
    row0 = s * ROWS_PER_SUB
    for blk in range(ROWS_PER_SUB // EB):
        pltpu.sync_copy(st0, acc_sh.at[pl.ds(row0 + blk * EB, EB)])
    pltpu.sync_copy(st0.at[pl.ds(0, ROWS_PER_SUB % EB)],
                    acc_sh.at[pl.ds(row0 + (ROWS_PER_SUB // EB) * EB,
                                    ROWS_PER_SUB % EB)])
    plsc.subcore_barrier()

    ebase = (c * NS + s) * E_PER_TILE

    def fire_idx(b):
        slot = lax.rem(b, 8)
        base = ebase + b * EB
        pltpu.async_copy(src_hbm.at[pl.ds(base, EB)], src_i.at[slot], sem_i)
        pltpu.async_copy(dst_hbm.at[pl.ds(base, EB)], dst_i.at[slot], sem_i)

    def drain_idx():
        pltpu.make_async_copy(src_hbm.at[pl.ds(0, EB)], src_i.at[0], sem_i).wait()
        pltpu.make_async_copy(dst_hbm.at[pl.ds(0, EB)], dst_i.at[0], sem_i).wait()

    def fire_gathers(b, p):
        slot = lax.rem(b, 8)
        base = ebase + b * EB
        pltpu.async_copy(qt_hbm.at[dst_i.at[slot]], qt_b[p], sem_g[p])
        pltpu.async_copy(kv_hbm.at[src_i.at[slot]], kv_b[p], sem_g[p])
        pltpu.async_copy(ea_hbm.at[pl.ds(base, EB)], ea_b[p], sem_g[p])

    def wait_gathers(b, p):
        slot = lax.rem(b, 8)
        base = ebase + b * EB
        pltpu.make_async_copy(qt_hbm.at[dst_i.at[slot]], qt_b[p], sem_g[p]).wait()
        pltpu.make_async_copy(kv_hbm.at[src_i.at[slot]], kv_b[p], sem_g[p]).wait()
        pltpu.make_async_copy(ea_hbm.at[pl.ds(base, EB)], ea_b[p], sem_g[p]).wait()

    def fire_scatter(b, p):
        slot = lax.rem(b, 8)
        base = ebase + b * EB
        pltpu.async_copy(st_b[p], acc_sh.at[dst_i.at[slot]], sem_c[p], add=True)
        pltpu.async_copy(sv_b[p].at[pl.ds(0, EB)], s_hbm.at[pl.ds(base, EB)],
                         sem_s[p])

    def drain_scatter(b_old, p):
        slot = lax.rem(b_old, 8)
        base = ebase + b_old * EB
        pltpu.make_async_copy(st_b[p], acc_sh.at[dst_i.at[slot]],
                              sem_c[p]).wait()
        pltpu.make_async_copy(sv_b[p].at[pl.ds(0, EB)],
                              s_hbm.at[pl.ds(base, EB)], sem_s[p]).wait()

    lane0 = lane == 0
    perm_even = jnp.where(lane < 8, lane * 2, 0)
    perm_odd = jnp.where(lane < 8, lane * 2 + 1, 0)
    col_even = 2 * lane
    col_odd = 2 * lane + 1

    def unpk(x):
        return plsc.unpack(x, format=plsc.PackFormat.INTERLEAVED,
                           preferred_element_type=jnp.float32)

    def compute(p):
        qt_v, kv_v, ea_v, st_v, sv_v = qt_b[p], kv_b[p], ea_b[p], st_b[p], sv_b[p]

        def body_e(e, _):
            acc = None
            for j in range(4):
                qa, qb = unpk(qt_v[e, pl.ds(32 * j, 32)])
                ka, kb = unpk(kv_v[e, pl.ds(32 * j, 32)])
                term = qa * ka + qb * kb
                acc = term if acc is None else acc + term
            te, to = unpk(qt_v[e, pl.ds(D, 32)])
            eav = ea_v[e, :]
            eae = jnp.take_along_axis(eav, perm_even, axis=0,
                                      mode="promise_in_bounds")
            eao = jnp.take_along_axis(eav, perm_odd, axis=0,
                                      mode="promise_in_bounds")
            acc = acc + te * eae + to * eao
            s_spl = jnp.exp(hsum_splat(acc) * INV_SQRT_C)
            erow = lax.broadcast(e, (16,))
            for j in range(4):
                va, vb = unpk(kv_v[e, pl.ds(D + 32 * j, 32)])
                plsc.store_scatter(st_v, [erow, 32 * j + col_even], s_spl * va)
                plsc.store_scatter(st_v, [erow, 32 * j + col_odd], s_spl * vb)
            plsc.store_scatter(sv_v, [erow], s_spl, mask=lane0)
            return 0

        lax.fori_loop(0, EB, body_e, 0)

    fire_idx(jnp.int32(0))
    fire_idx(jnp.int32(1))
    fire_idx(jnp.int32(2))
    fire_idx(jnp.int32(3))
    drain_idx()
    drain_idx()
    fire_gathers(jnp.int32(0), 0)
    fire_gathers(jnp.int32(1), 1)

    def loop_body(i, _):
        for p in (0, 1):
            b = 2 * i + p
            wait_gathers(b, p)

            @pl.when(b >= 2)
            def _():
                drain_scatter(jnp.maximum(b - 2, 0), p)

            compute(p)
            fire_scatter(b, p)

            @pl.when(b + 2 < NBLK)
            def _():
                drain_idx()
                fire_gathers(b + 2, p)

            @pl.when(b + 4 < NBLK)
            def _():
                fire_idx(b + 4)
        return 0

    lax.fori_loop(0, NBLK // 2, loop_body, 0)
    drain_scatter(jnp.int32(NBLK - 2), 0)
    drain_scatter(jnp.int32(NBLK - 1), 1)
    plsc.subcore_barrier()

    for blk in range(ROWS_PER_SUB // EB):
        rows = pl.ds(row0 + blk * EB, EB)
        pltpu.sync_copy(acc_sh.at[rows], st0)
        pltpu.sync_copy(st0, acc_hbm.at[c, rows])
    rem = ROWS_PER_SUB % EB
    rows = pl.ds(row0 + (ROWS_PER_SUB // EB) * EB, rem)
    pltpu.sync_copy(acc_sh.at[rows], st0.at[pl.ds(0, rem)])
    pltpu.sync_copy(st0.at[pl.ds(0, rem)], acc_hbm.at[c, rows])


@functools.cache
def _sc1():
    return pl.kernel(
        _sc1_body,
        out_type=(jax.ShapeDtypeStruct((NC, N, D), jnp.float32),
                  jax.ShapeDtypeStruct((E,), jnp.float32)),
        mesh=plsc.VectorSubcoreMesh(core_axis_name="c", subcore_axis_name="s",
                                    num_cores=NC, num_subcores=NS),
        compiler_params=pltpu.CompilerParams(use_tc_tiling_on_sc=False, needs_layout_passes=False),
        scratch_types=[
            pltpu.VMEM((8, EB), jnp.int32),
            pltpu.VMEM((8, EB), jnp.int32),
            pltpu.VMEM((EB, QTB), jnp.bfloat16),
            pltpu.VMEM((EB, QTB), jnp.bfloat16),
            pltpu.VMEM((EB, 2 * D), jnp.bfloat16),
            pltpu.VMEM((EB, 2 * D), jnp.bfloat16),
            pltpu.VMEM((EB, DE), jnp.float32),
            pltpu.VMEM((EB, DE), jnp.float32),
            pltpu.VMEM((EB, D), jnp.float32),
            pltpu.VMEM((EB, D), jnp.float32),
            pltpu.VMEM((48,), jnp.float32),
            pltpu.VMEM((48,), jnp.float32),
            pltpu.VMEM_SHARED((N, D), jnp.float32),
            pltpu.SemaphoreType.DMA,
            pltpu.SemaphoreType.DMA,
            pltpu.SemaphoreType.DMA,
            pltpu.SemaphoreType.DMA,
            pltpu.SemaphoreType.DMA,
            pltpu.SemaphoreType.DMA,
            pltpu.SemaphoreType.DMA,
        ],
    )


def _sc2_body(s_hbm, ea_hbm, dst_hbm, acc_hbm,
              dst_i, s0, s1, ea0, ea1, st0, st1,
              acc_sh, sem_g0, sem_g1, sem_s0, sem_s1):
    c = lax.axis_index("c")
    s = lax.axis_index("s")
    s_b = (s0, s1)
    ea_b = (ea0, ea1)
    st_b = (st0, st1)
    sem_g = (sem_g0, sem_g1)
    sem_s = (sem_s0, sem_s1)

    zeros16 = jnp.zeros((16,), jnp.float32)
    lane = lax.iota(jnp.int32, 16)
    lane0 = lane == 0

    def zero_row(r, _):
        st0[r, pl.ds(0, 16)] = zeros16
        st0[r, pl.ds(16, 16)] = zeros16
        return 0

    lax.fori_loop(0, EB2, zero_row, 0)
    row0 = s * ROWS_PER_SUB
    for blk in range(ROWS_PER_SUB // EB2):
        pltpu.sync_copy(st0, acc_sh.at[pl.ds(row0 + blk * EB2, EB2)])
    pltpu.sync_copy(st0.at[pl.ds(0, ROWS_PER_SUB % EB2)],
                    acc_sh.at[pl.ds(row0 + (ROWS_PER_SUB // EB2) * EB2,
                                    ROWS_PER_SUB % EB2)])
    plsc.subcore_barrier()

    ebase = (c * NS + s) * E_PER_TILE

    def fire_in(b, p):
        slot = lax.rem(b, 4)
        base = ebase + b * EB2
        pltpu.async_copy(s_hbm.at[pl.ds(base, EB2)], s_b[p], sem_g[p])
        pltpu.async_copy(ea_hbm.at[pl.ds(base, EB2)], ea_b[p], sem_g[p])
        pltpu.async_copy(dst_hbm.at[pl.ds(base, EB2)], dst_i.at[slot], sem_g[p])

    def wait_in(b, p):
        slot = lax.rem(b, 4)
        base = ebase + b * EB2
        pltpu.make_async_copy(s_hbm.at[pl.ds(base, EB2)], s_b[p], sem_g[p]).wait()
        pltpu.make_async_copy(ea_hbm.at[pl.ds(base, EB2)], ea_b[p], sem_g[p]).wait()
        pltpu.make_async_copy(dst_hbm.at[pl.ds(base, EB2)], dst_i.at[slot],
                              sem_g[p]).wait()

    def fire_scatter(b, p):
        slot = lax.rem(b, 4)
        pltpu.async_copy(st_b[p], acc_sh.at[dst_i.at[slot]], sem_s[p], add=True)

    def drain_scatter(b_old, p):
        slot = lax.rem(b_old, 4)
        pltpu.make_async_copy(st_b[p], acc_sh.at[dst_i.at[slot]],
                              sem_s[p]).wait()

    def compute(p):
        s_v, ea_v, st_v = s_b[p], ea_b[p], st_b[p]

        def body_e(e, _):
            idx = lax.broadcast(e, (16,))
            s_spl = plsc.load_gather(s_v, [idx])
            st_v[e, pl.ds(0, 16)] = jnp.where(lane0, s_spl, 0.0)
            st_v[e, pl.ds(16, 16)] = s_spl * ea_v[e, :]
            return 0

        lax.fori_loop(0, EB2, body_e, 0)

    fire_in(jnp.int32(0), 0)
    fire_in(jnp.int32(1), 1)

    def loop_body(i, _):
        for p in (0, 1):
            b = 2 * i + p
            wait_in(b, p)

            @pl.when(b >= 2)
            def _():
                drain_scatter(jnp.maximum(b - 2, 0), p)

            compute(p)
            fire_scatter(b, p)

            @pl.when(b + 2 < NBLK2)
            def _():
                fire_in(b + 2, p)
        return 0

    lax.fori_loop(0, NBLK2 // 2, loop_body, 0)
    drain_scatter(jnp.int32(NBLK2 - 2), 0)
    drain_scatter(jnp.int32(NBLK2 - 1), 1)
    plsc.subcore_barrier()

    for blk in range(ROWS_PER_SUB // EB2):
        rows = pl.ds(row0 + blk * EB2, EB2)
        pltpu.sync_copy(acc_sh.at[rows], st0)
        pltpu.sync_copy(st0, acc_hbm.at[c, rows])
    rem = ROWS_PER_SUB % EB2
    rows = pl.ds(row0 + (ROWS_PER_SUB // EB2) * EB2, rem)
    pltpu.sync_copy(acc_sh.at[rows], st0.at[pl.ds(0, rem)])
    pltpu.sync_copy(st0.at[pl.ds(0, rem)], acc_hbm.at[c, rows])


@functools.cache
def _sc2():
    return pl.kernel(
        _sc2_body,
        out_type=jax.ShapeDtypeStruct((NC, N, A2W), jnp.float32),
        mesh=plsc.VectorSubcoreMesh(core_axis_name="c", subcore_axis_name="s",
                                    num_cores=NC, num_subcores=NS),
        compiler_params=pltpu.CompilerParams(use_tc_tiling_on_sc=False, needs_layout_passes=False),
        scratch_types=[
            pltpu.VMEM((4, EB2), jnp.int32),
            pltpu.VMEM((EB2,), jnp.float32),
            pltpu.VMEM((EB2,), jnp.float32),
            pltpu.VMEM((EB2, DE), jnp.float32),
            pltpu.VMEM((EB2, DE), jnp.float32),
            pltpu.VMEM((EB2, A2W), jnp.float32),
            pltpu.VMEM((EB2, A2W), jnp.float32),
            pltpu.VMEM_SHARED((N, A2W), jnp.float32),
            pltpu.SemaphoreType.DMA,
            pltpu.SemaphoreType.DMA,
            pltpu.SemaphoreType.DMA,
            pltpu.SemaphoreType.DMA,
        ],
    )


def _post_body(m0_r, m1_r, d0_r, d1_r, sk_r, we_r, out_r):
    msg = m0_r[...] + m1_r[...]
    d = d0_r[...] + d1_r[...]
    denom = d[:, 0:1] + 1e-16
    sea = d[:, DE:2 * DE]
    msg = (msg + jnp.dot(sea, we_r[...], preferred_element_type=jnp.float32)) / denom
    out_r[...] = jnp.maximum(msg + sk_r[...], 0.0)


def _tc_post(m0, m1, d0, d1, sk, We):
    return pl.pallas_call(
        _post_body,
        grid=(_GRID,),
        in_specs=[
            pl.BlockSpec((_R, D), lambda i: (i, 0)),
            pl.BlockSpec((_R, D), lambda i: (i, 0)),
            pl.BlockSpec((_R, A2W), lambda i: (i, 0)),
            pl.BlockSpec((_R, A2W), lambda i: (i, 0)),
            pl.BlockSpec((_R, C), lambda i: (i, 0)),
            pl.BlockSpec((DE, C), lambda i: (0, 0)),
        ],
        out_specs=pl.BlockSpec((_R, C), lambda i: (i, 0)),
        out_shape=jax.ShapeDtypeStruct((N, C), jnp.float32),
    )(m0, m1, d0, d1, sk, We)


@jax.jit
def kernel(x, edge_index, edge_attr, Wq, bq, Wk, bk, Wv, bv, We, Wskip, bskip):
    src = edge_index[0]
    dst = edge_index[1]
    qt, kv, sk = _tc_pre(
        x, Wq, bq.reshape(1, C), Wk, bk.reshape(1, C), Wv, bv.reshape(1, C),
        We.T, Wskip, bskip.reshape(1, C))
    msg, s_e = _sc1()(qt, kv, edge_attr, src, dst)
    dsea = _sc2()(s_e, edge_attr, dst)
    return _tc_post(msg[0], msg[1], dsea[0], dsea[1], sk, We)


# speedup vs baseline: 1.0246x; 1.0246x over previous
import functools

import jax
import jax.numpy as jnp
from jax import lax
from jax.experimental import pallas as pl
from jax.experimental.pallas import tpu as pltpu
from jax.experimental.pallas import tpu_sc as plsc

N = 10000
E = 320000
D = 128
DE = 16
C = 128

NC = 2
NS = 16
NW = NC * NS
EB = 40
EB2 = 200
E_PER_TILE = E // NW
NBLK = E_PER_TILE // EB
NBLK2 = E_PER_TILE // EB2
ROWS_PER_SUB = 625
QTB = 160
A2W = 32
INV_SQRT_C = 1.0 / (C ** 0.5)

_R = 400
_GRID = N // _R


def _pre_body(x_r, wq_r, bq_r, wk_r, bk_r, wv_r, bv_r, wet_r, wsk_r, bsk_r,
              qt_r, kv_r, sk_r):
    xb = x_r[...]
    q = jnp.dot(xb, wq_r[...], preferred_element_type=jnp.float32) + bq_r[...]
    qt_r[:, :D] = q.astype(jnp.bfloat16)
    qt_r[:, D:D + DE] = jnp.dot(
        q, wet_r[...], preferred_element_type=jnp.float32).astype(jnp.bfloat16)
    qt_r[:, D + DE:] = jnp.zeros((_R, QTB - D - DE), jnp.bfloat16)
    kv_r[:, :D] = (jnp.dot(xb, wk_r[...], preferred_element_type=jnp.float32)
                   + bk_r[...]).astype(jnp.bfloat16)
    kv_r[:, D:] = (jnp.dot(xb, wv_r[...], preferred_element_type=jnp.float32)
                   + bv_r[...]).astype(jnp.bfloat16)
    sk_r[...] = jnp.dot(xb, wsk_r[...], preferred_element_type=jnp.float32) + bsk_r[...]


def _tc_pre(x, Wq, bq, Wk, bk, Wv, bv, WeT, Wskip, bskip):
    full = lambda shape: pl.BlockSpec(shape, lambda i: (0,) * len(shape))
    return pl.pallas_call(
        _pre_body,
        grid=(_GRID,),
        in_specs=[
            pl.BlockSpec((_R, D), lambda i: (i, 0)),
            full((D, C)), full((1, C)),
            full((D, C)), full((1, C)),
            full((D, C)), full((1, C)),
            full((D, DE)),
            full((D, C)), full((1, C)),
        ],
        out_specs=[
            pl.BlockSpec((_R, QTB), lambda i: (i, 0)),
            pl.BlockSpec((_R, 2 * D), lambda i: (i, 0)),
            pl.BlockSpec((_R, C), lambda i: (i, 0)),
        ],
        out_shape=[
            jax.ShapeDtypeStruct((N, QTB), jnp.bfloat16),
            jax.ShapeDtypeStruct((N, 2 * D), jnp.bfloat16),
            jax.ShapeDtypeStruct((N, C), jnp.float32),
        ],
    )(x, Wq, bq, Wk, bk, Wv, bv, WeT, Wskip, bskip)


def _sc1_body(qt_hbm, kv_hbm, ea_hbm, src_hbm, dst_hbm, acc_hbm, s_hbm,
              src_i, dst_i, qt0, qt1, kv0, kv1, ea0, ea1, st0, st1, sv0, sv1,
              acc_sh, sem_g0, sem_g1, sem_s0, sem_s1, sem_c0, sem_c1, sem_i):
    c = lax.axis_index("c")
    s = lax.axis_index("s")
    qt_b = (qt0, qt1)
    kv_b = (kv0, kv1)
    ea_b = (ea0, ea1)
    st_b = (st0, st1)
    sv_b = (sv0, sv1)
    sem_g = (sem_g0, sem_g1)
    sem_s = (sem_s0, sem_s1)
    sem_c = (sem_c0, sem_c1)

    zeros16 = jnp.zeros((16,), jnp.float32)
    lane = lax.iota(jnp.int32, 16)
    perms = [lane ^ m for m in (8, 4, 2, 1)]

    def hsum_splat(vec):
        for p in perms:
            vec = vec + jnp.take_along_axis(vec, p, axis=0,
                                            mode="promise_in_bounds")
        return vec

    def zero_row(r, _):
        for jj in range(D // 16):
            st0[r, pl.ds(16 * jj, 16)] = zeros16
        return 0

    lax.fori_loop(0, EB, zero_row, 0)
    row0 = s * ROWS_PER_SUB
    for blk in range(ROWS_PER_SUB // EB):
        pltpu.sync_copy(st0, acc_sh.at[pl.ds(row0 + blk * EB, EB)])
    pltpu.sync_copy(st0.at[pl.ds(0, ROWS_PER_SUB % EB)],
                    acc_sh.at[pl.ds(row0 + (ROWS_PER_SUB // EB) * EB,
                                    ROWS_PER_SUB % EB)])
    plsc.subcore_barrier()

    ebase = (c * NS + s) * E_PER_TILE

    def fire_idx(b):
        slot = lax.rem(b, 8)
        base = ebase + b * EB
        pltpu.async_copy(src_hbm.at[pl.ds(base, EB)], src_i.at[slot], sem_i)
        pltpu.async_copy(dst_hbm.at[pl.ds(base, EB)], dst_i.at[slot], sem_i)

    def drain_idx():
        pltpu.make_async_copy(src_hbm.at[pl.ds(0, EB)], src_i.at[0], sem_i).wait()
        pltpu.make_async_copy(dst_hbm.at[pl.ds(0, EB)], dst_i.at[0], sem_i).wait()

    def fire_gathers(b, p):
        slot = lax.rem(b, 8)
        base = ebase + b * EB
        pltpu.async_copy(qt_hbm.at[dst_i.at[slot]], qt_b[p], sem_g[p])
        pltpu.async_copy(kv_hbm.at[src_i.at[slot]], kv_b[p], sem_g[p])
        pltpu.async_copy(ea_hbm.at[pl.ds(base, EB)], ea_b[p], sem_g[p])

    def wait_gathers(b, p):
        slot = lax.rem(b, 8)
        base = ebase + b * EB
        pltpu.make_async_copy(qt_hbm.at[dst_i.at[slot]], qt_b[p], sem_g[p]).wait()
        pltpu.make_async_copy(kv_hbm.at[src_i.at[slot]], kv_b[p], sem_g[p]).wait()
        pltpu.make_async_copy(ea_hbm.at[pl.ds(base, EB)], ea_b[p], sem_g[p]).wait()

    def fire_scatter(b, p):
        slot = lax.rem(b, 8)
        base = ebase + b * EB
        pltpu.async_copy(st_b[p], acc_sh.at[dst_i.at[slot]], sem_c[p], add=True)
        pltpu.async_copy(sv_b[p].at[pl.ds(0, EB)], s_hbm.at[pl.ds(base, EB)],
                         sem_s[p])

    def drain_scatter(b_old, p):
        slot = lax.rem(b_old, 8)
        base = ebase + b_old * EB
        pltpu.make_async_copy(st_b[p], acc_sh.at[dst_i.at[slot]],
                              sem_c[p]).wait()
        pltpu.make_async_copy(sv_b[p].at[pl.ds(0, EB)],
                              s_hbm.at[pl.ds(base, EB)], sem_s[p]).wait()

    lane0 = lane == 0
    perm_even = jnp.where(lane < 8, lane * 2, 0)
    perm_odd = jnp.where(lane < 8, lane * 2 + 1, 0)
    col_even = 2 * lane
    col_odd = 2 * lane + 1

    def unpk(x):
        return plsc.unpack(x, format=plsc.PackFormat.INTERLEAVED,
                           preferred_element_type=jnp.float32)

    def compute(p):
        qt_v, kv_v, ea_v, st_v, sv_v = qt_b[p], kv_b[p], ea_b[p], st_b[p], sv_b[p]

        def body_e(e, _):
            acc = None
            for j in range(4):
                qa, qb = unpk(qt_v[e, pl.ds(32 * j, 32)])
                ka, kb = unpk(kv_v[e, pl.ds(32 * j, 32)])
                term = qa * ka + qb * kb
                acc = term if acc is None else acc + term
            te, to = unpk(qt_v[e, pl.ds(D, 32)])
            eav = ea_v[e, :]
            eae = jnp.take_along_axis(eav, perm_even, axis=0,
                                      mode="promise_in_bounds")
            eao = jnp.take_along_axis(eav, perm_odd, axis=0,
                                      mode="promise_in_bounds")
            acc = acc + te * eae + to * eao
            s_spl = jnp.exp(hsum_splat(acc) * INV_SQRT_C)
            erow = lax.broadcast(e, (16,))
            for j in range(4):
                va, vb = unpk(kv_v[e, pl.ds(D + 32 * j, 32)])
                plsc.store_scatter(st_v, [erow, 32 * j + col_even], s_spl * va)
                plsc.store_scatter(st_v, [erow, 32 * j + col_odd], s_spl * vb)
            plsc.store_scatter(sv_v, [erow], s_spl, mask=lane0)
            return 0

        lax.fori_loop(0, EB, body_e, 0)

    fire_idx(jnp.int32(0))
    fire_idx(jnp.int32(1))
    fire_idx(jnp.int32(2))
    fire_idx(jnp.int32(3))
    drain_idx()
    drain_idx()
    fire_gathers(jnp.int32(0), 0)
    fire_gathers(jnp.int32(1), 1)

    def loop_body(i, _):
        for p in (0, 1):
            b = 2 * i + p
            wait_gathers(b, p)

            @pl.when(b >= 2)
            def _():
                drain_scatter(jnp.maximum(b - 2, 0), p)

            compute(p)
            fire_scatter(b, p)

            @pl.when(b + 2 < NBLK)
            def _():
                drain_idx()
                fire_gathers(b + 2, p)

            @pl.when(b + 4 < NBLK)
            def _():
                fire_idx(b + 4)
        return 0

    lax.fori_loop(0, NBLK // 2, loop_body, 0)
    drain_scatter(jnp.int32(NBLK - 2), 0)
    drain_scatter(jnp.int32(NBLK - 1), 1)
    plsc.subcore_barrier()

    for blk in range(ROWS_PER_SUB // EB):
        rows = pl.ds(row0 + blk * EB, EB)
        pltpu.sync_copy(acc_sh.at[rows], st0)
        pltpu.sync_copy(st0, acc_hbm.at[c, rows])
    rem = ROWS_PER_SUB % EB
    rows = pl.ds(row0 + (ROWS_PER_SUB // EB) * EB, rem)
    pltpu.sync_copy(acc_sh.at[rows], st0.at[pl.ds(0, rem)])
    pltpu.sync_copy(st0.at[pl.ds(0, rem)], acc_hbm.at[c, rows])


@functools.cache
def _sc1():
    return pl.kernel(
        _sc1_body,
        out_type=(jax.ShapeDtypeStruct((NC, N, D), jnp.float32),
                  jax.ShapeDtypeStruct((E,), jnp.float32)),
        mesh=plsc.VectorSubcoreMesh(core_axis_name="c", subcore_axis_name="s",
                                    num_cores=NC, num_subcores=NS),
        compiler_params=pltpu.CompilerParams(use_tc_tiling_on_sc=False, needs_layout_passes=False),
        scratch_types=[
            pltpu.VMEM((8, EB), jnp.int32),
            pltpu.VMEM((8, EB), jnp.int32),
            pltpu.VMEM((EB, QTB), jnp.bfloat16),
            pltpu.VMEM((EB, QTB), jnp.bfloat16),
            pltpu.VMEM((EB, 2 * D), jnp.bfloat16),
            pltpu.VMEM((EB, 2 * D), jnp.bfloat16),
            pltpu.VMEM((EB, DE), jnp.float32),
            pltpu.VMEM((EB, DE), jnp.float32),
            pltpu.VMEM((EB, D), jnp.float32),
            pltpu.VMEM((EB, D), jnp.float32),
            pltpu.VMEM((48,), jnp.float32),
            pltpu.VMEM((48,), jnp.float32),
            pltpu.VMEM_SHARED((N, D), jnp.float32),
            pltpu.SemaphoreType.DMA,
            pltpu.SemaphoreType.DMA,
            pltpu.SemaphoreType.DMA,
            pltpu.SemaphoreType.DMA,
            pltpu.SemaphoreType.DMA,
            pltpu.SemaphoreType.DMA,
            pltpu.SemaphoreType.DMA,
        ],
    )


def _sc2_body(s_hbm, ea_hbm, dst_hbm, acc_hbm,
              dst_i, s0, s1, ea0, ea1, st0, st1,
              acc_sh, sem_g0, sem_g1, sem_s0, sem_s1):
    c = lax.axis_index("c")
    s = lax.axis_index("s")
    s_b = (s0, s1)
    ea_b = (ea0, ea1)
    st_b = (st0, st1)
    sem_g = (sem_g0, sem_g1)
    sem_s = (sem_s0, sem_s1)

    zeros16 = jnp.zeros((16,), jnp.float32)
    lane = lax.iota(jnp.int32, 16)
    lane0 = lane == 0

    def zero_row(r, _):
        st0[r, pl.ds(0, 16)] = zeros16
        st0[r, pl.ds(16, 16)] = zeros16
        return 0

    lax.fori_loop(0, EB2, zero_row, 0)
    row0 = s * ROWS_PER_SUB
    for blk in range(ROWS_PER_SUB // EB2):
        pltpu.sync_copy(st0, acc_sh.at[pl.ds(row0 + blk * EB2, EB2)])
    pltpu.sync_copy(st0.at[pl.ds(0, ROWS_PER_SUB % EB2)],
                    acc_sh.at[pl.ds(row0 + (ROWS_PER_SUB // EB2) * EB2,
                                    ROWS_PER_SUB % EB2)])
    plsc.subcore_barrier()

    ebase = (c * NS + s) * E_PER_TILE

    def fire_in(b, p):
        slot = lax.rem(b, 4)
        base = ebase + b * EB2
        pltpu.async_copy(s_hbm.at[pl.ds(base, EB2)], s_b[p], sem_g[p])
        pltpu.async_copy(ea_hbm.at[pl.ds(base, EB2)], ea_b[p], sem_g[p])
        pltpu.async_copy(dst_hbm.at[pl.ds(base, EB2)], dst_i.at[slot], sem_g[p])

    def wait_in(b, p):
        slot = lax.rem(b, 4)
        base = ebase + b * EB2
        pltpu.make_async_copy(s_hbm.at[pl.ds(base, EB2)], s_b[p], sem_g[p]).wait()
        pltpu.make_async_copy(ea_hbm.at[pl.ds(base, EB2)], ea_b[p], sem_g[p]).wait()
        pltpu.make_async_copy(dst_hbm.at[pl.ds(base, EB2)], dst_i.at[slot],
                              sem_g[p]).wait()

    def fire_scatter(b, p):
        slot = lax.rem(b, 4)
        pltpu.async_copy(st_b[p], acc_sh.at[dst_i.at[slot]], sem_s[p], add=True)

    def drain_scatter(b_old, p):
        slot = lax.rem(b_old, 4)
        pltpu.make_async_copy(st_b[p], acc_sh.at[dst_i.at[slot]],
                              sem_s[p]).wait()

    def compute(p):
        s_v, ea_v, st_v = s_b[p], ea_b[p], st_b[p]

        def body_e(e, _):
            idx = lax.broadcast(e, (16,))
            s_spl = plsc.load_gather(s_v, [idx])
            st_v[e, pl.ds(0, 16)] = jnp.where(lane0, s_spl, 0.0)
            st_v[e, pl.ds(16, 16)] = s_spl * ea_v[e, :]
            return 0

        lax.fori_loop(0, EB2, body_e, 0)

    fire_in(jnp.int32(0), 0)
    fire_in(jnp.int32(1), 1)

    def loop_body(i, _):
        for p in (0, 1):
            b = 2 * i + p
            wait_in(b, p)

            @pl.when(b >= 2)
            def _():
                drain_scatter(jnp.maximum(b - 2, 0), p)

            compute(p)
            fire_scatter(b, p)

            @pl.when(b + 2 < NBLK2)
            def _():
                fire_in(b + 2, p)
        return 0

    lax.fori_loop(0, NBLK2 // 2, loop_body, 0)
    drain_scatter(jnp.int32(NBLK2 - 2), 0)
    drain_scatter(jnp.int32(NBLK2 - 1), 1)
    plsc.subcore_barrier()

    for blk in range(ROWS_PER_SUB // EB2):
        rows = pl.ds(row0 + blk * EB2, EB2)
        pltpu.sync_copy(acc_sh.at[rows], st0)
        pltpu.sync_copy(st0, acc_hbm.at[c, rows])
    rem = ROWS_PER_SUB % EB2
    rows = pl.ds(row0 + (ROWS_PER_SUB // EB2) * EB2, rem)
    pltpu.sync_copy(acc_sh.at[rows], st0.at[pl.ds(0, rem)])
    pltpu.sync_copy(st0.at[pl.ds(0, rem)], acc_hbm.at[c, rows])


@functools.cache
def _sc2():
    return pl.kernel(
        _sc2_body,
        out_type=jax.ShapeDtypeStruct((NC, N, A2W), jnp.float32),
        mesh=plsc.VectorSubcoreMesh(core_axis_name="c", subcore_axis_name="s",
                                    num_cores=NC, num_subcores=NS),
        compiler_params=pltpu.CompilerParams(use_tc_tiling_on_sc=False, needs_layout_passes=False),
        scratch_types=[
            pltpu.VMEM((4, EB2), jnp.int32),
            pltpu.VMEM((EB2,), jnp.float32),
            pltpu.VMEM((EB2,), jnp.float32),
            pltpu.VMEM((EB2, DE), jnp.float32),
            pltpu.VMEM((EB2, DE), jnp.float32),
            pltpu.VMEM((EB2, A2W), jnp.float32),
            pltpu.VMEM((EB2, A2W), jnp.float32),
            pltpu.VMEM_SHARED((N, A2W), jnp.float32),
            pltpu.SemaphoreType.DMA,
            pltpu.SemaphoreType.DMA,
            pltpu.SemaphoreType.DMA,
            pltpu.SemaphoreType.DMA,
        ],
    )


def _post_body(m0_r, m1_r, d0_r, d1_r, sk_r, we_r, out_r):
    msg = m0_r[...] + m1_r[...]
    d = d0_r[...] + d1_r[...]
    denom = d[:, 0:1] + 1e-16
    sea = d[:, DE:2 * DE]
    msg = (msg + jnp.dot(sea, we_r[...], preferred_element_type=jnp.float32)) / denom
    out_r[...] = jnp.maximum(msg + sk_r[...], 0.0)


def _tc_post(m0, m1, d0, d1, sk, We):
    return pl.pallas_call(
        _post_body,
        grid=(_GRID,),
        in_specs=[
            pl.BlockSpec((_R, D), lambda i: (i, 0)),
            pl.BlockSpec((_R, D), lambda i: (i, 0)),
            pl.BlockSpec((_R, A2W), lambda i: (i, 0)),
            pl.BlockSpec((_R, A2W), lambda i: (i, 0)),
            pl.BlockSpec((_R, C), lambda i: (i, 0)),
            pl.BlockSpec((DE, C), lambda i: (0, 0)),
        ],
        out_specs=pl.BlockSpec((_R, C), lambda i: (i, 0)),
        out_shape=jax.ShapeDtypeStruct((N, C), jnp.float32),
    )(m0, m1, d0, d1, sk, We)


@jax.jit
def kernel(x, edge_index, edge_attr, Wq, bq, Wk, bk, Wv, bv, We, Wskip, bskip):
    src = edge_index[0]
    dst = edge_index[1]
    qt, kv, sk = _tc_pre(
        x, Wq, bq.reshape(1, C), Wk, bk.reshape(1, C), Wv, bv.reshape(1, C),
        We.T, Wskip, bskip.reshape(1, C))
    msg, s_e = _sc1()(qt, kv, edge_attr, src, dst)
    dsea = _sc2()(s_e, edge_attr, dst)
    return _tc_post(msg[0], msg[1], dsea[0], dsea[1], sk, We)
